# baseline (speedup 1.0000x reference)
import functools

import jax
import jax.numpy as jnp
from jax import lax
from jax.experimental import pallas as pl
from jax.experimental.pallas import tpu as pltpu
from jax.experimental.pallas import tpu_sc as plsc

D = 32
H = 16
B = 16384
L = 16
NC, NS = 2, 16
NW = NC * NS
BPW = B // NW
CHUNK = 128
NCHUNK = BPW // CHUNK
NBLK = BPW // L


def _gmf_body(cell_idx_hbm, gene_idx_hbm, cell_tab, gene_tab,
              w1b_hbm, b1b_hbm, w2b_hbm, b2b_hbm, out_hbm,
              idx_c, idx_g, cell_v, gene_v, w1v, b1v, w2v, b2v, out_v, sem):
    wid = lax.axis_index("s") * NC + lax.axis_index("c")
    base = wid * BPW

    pltpu.sync_copy(cell_idx_hbm.at[pl.ds(base, BPW)], idx_c)
    pltpu.sync_copy(gene_idx_hbm.at[pl.ds(base, BPW)], idx_g)
    pltpu.sync_copy(w1b_hbm, w1v)
    pltpu.sync_copy(b1b_hbm, b1v)
    pltpu.sync_copy(w2b_hbm, w2v)
    pltpu.sync_copy(b2b_hbm, b2v)

    copies = []
    for c in range(NCHUNK):
        sl = pl.ds(c * CHUNK, CHUNK)
        copies.append(pltpu.async_copy(cell_tab.at[idx_c.at[sl]],
                                       cell_v.at[sl], sem))
        copies.append(pltpu.async_copy(gene_tab.at[idx_g.at[sl]],
                                       gene_v.at[sl], sem))
    for cp in copies:
        cp.wait()

    lanes = lax.iota(jnp.int32, L)

    def blk_body(blk, carry):
        rows = lanes + blk * L
        h = [b1v[pl.ds(k * L, L)] for k in range(H)]
        for d in range(D):
            dcol = jnp.full((L,), d, jnp.int32)
            pc = plsc.load_gather(cell_v, [rows, dcol])
            pg = plsc.load_gather(gene_v, [rows, dcol])
            p = pc * pg
            for k in range(H):
                h[k] = h[k] + p * w1v[pl.ds((d * H + k) * L, L)]
        acc = b2v[pl.ds(0, L)]
        for k in range(H):
            acc = acc + jnp.maximum(h[k], 0.0) * w2v[pl.ds(k * L, L)]
        out_v[pl.ds(blk * L, L)] = 1.0 / (1.0 + jnp.exp(-acc))
        return carry

    lax.fori_loop(0, NBLK, blk_body, 0)

    pltpu.sync_copy(out_v, out_hbm.at[pl.ds(base, BPW)])


@functools.partial(
    pl.kernel,
    out_type=jax.ShapeDtypeStruct((B,), jnp.float32),
    mesh=plsc.VectorSubcoreMesh(core_axis_name="c", subcore_axis_name="s"),
    compiler_params=pltpu.CompilerParams(needs_layout_passes=False,
                                         use_tc_tiling_on_sc=False),
    scratch_types=[
        pltpu.VMEM((BPW,), jnp.int32),
        pltpu.VMEM((BPW,), jnp.int32),
        pltpu.VMEM((BPW, D), jnp.float32),
        pltpu.VMEM((BPW, D), jnp.float32),
        pltpu.VMEM((D * H * L,), jnp.float32),
        pltpu.VMEM((H * L,), jnp.float32),
        pltpu.VMEM((H * L,), jnp.float32),
        pltpu.VMEM((L,), jnp.float32),
        pltpu.VMEM((BPW,), jnp.float32),
        pltpu.SemaphoreType.DMA,
    ],
)
def _gmf_kernel(*refs):
    _gmf_body(*refs)


def kernel(cell_indices, gene_indices, emb_cell, emb_gene, W1, b1, W2, b2):
    w1b = jnp.broadcast_to(W1.reshape(D, H, 1), (D, H, L)).reshape(-1)
    b1b = jnp.broadcast_to(b1.reshape(H, 1), (H, L)).reshape(-1)
    w2b = jnp.broadcast_to(W2.reshape(H, 1), (H, L)).reshape(-1)
    b2b = jnp.broadcast_to(b2.reshape(1, 1), (1, L)).reshape(-1)
    out = _gmf_kernel(cell_indices.astype(jnp.int32),
                      gene_indices.astype(jnp.int32),
                      emb_cell, emb_gene, w1b, b1b, w2b, b2b)
    return out.reshape(B, 1)


# speedup vs baseline: 1.0452x; 1.0452x over previous
import functools

import jax
import jax.numpy as jnp
from jax import lax
from jax.experimental import pallas as pl
from jax.experimental.pallas import tpu as pltpu
from jax.experimental.pallas import tpu_sc as plsc

D = 32
H = 16
B = 16384
L = 16
NC, NS = 2, 16
NW = NC * NS
BPW = B // NW
SB = 256
NSB = BPW // SB
CHUNK = 128
NCHUNK = SB // CHUNK
NBLK = SB // L


def _gmf_body(cell_idx_hbm, gene_idx_hbm, cell_tab, gene_tab,
              w1b_hbm, b1b_hbm, w2b_hbm, b2b_hbm, out_hbm,
              idx_c, idx_g, slot_c, slot_g, col_c, col_g,
              cell_v, gene_v, w1v, b1v, w2v, b2v, out_v, sem):
    wid = lax.axis_index("s") * NC + lax.axis_index("c")
    base = wid * BPW

    pltpu.sync_copy(cell_idx_hbm.at[pl.ds(base, BPW)], idx_c)
    pltpu.sync_copy(gene_idx_hbm.at[pl.ds(base, BPW)], idx_g)
    pltpu.sync_copy(w1b_hbm, w1v)
    pltpu.sync_copy(b1b_hbm, b1v)
    pltpu.sync_copy(w2b_hbm, w2v)
    pltpu.sync_copy(b2b_hbm, b2v)

    for i in range(BPW // L):
        sl = pl.ds(i * L, L)
        ic = idx_c[sl]
        ig = idx_g[sl]
        slot_c[sl] = lax.shift_right_logical(ic, 2)
        slot_g[sl] = lax.shift_right_logical(ig, 2)
        col_c[sl] = lax.shift_left(jnp.bitwise_and(ic, 3), 5)
        col_g[sl] = lax.shift_left(jnp.bitwise_and(ig, 3), 5)

    lanes = lax.iota(jnp.int32, L)

    for sb in range(NSB):
        copies = []
        for c in range(NCHUNK):
            src = pl.ds(sb * SB + c * CHUNK, CHUNK)
            dst = pl.ds(c * CHUNK, CHUNK)
            copies.append(pltpu.async_copy(cell_tab.at[slot_c.at[src]],
                                           cell_v.at[dst], sem))
            copies.append(pltpu.async_copy(gene_tab.at[slot_g.at[src]],
                                           gene_v.at[dst], sem))
        for cp in copies:
            cp.wait()

        def blk_body(blk, carry):
            rows = lanes + blk * L
            cbase = col_c[pl.ds(sb * SB + blk * L, L)]
            gbase = col_g[pl.ds(sb * SB + blk * L, L)]
            h = [b1v[pl.ds(k * L, L)] for k in range(H)]
            for d in range(D):
                pc = plsc.load_gather(cell_v, [rows, cbase + d])
                pg = plsc.load_gather(gene_v, [rows, gbase + d])
                p = pc * pg
                for k in range(H):
                    h[k] = h[k] + p * w1v[pl.ds((d * H + k) * L, L)]
            acc = b2v[pl.ds(0, L)]
            for k in range(H):
                acc = acc + jnp.maximum(h[k], 0.0) * w2v[pl.ds(k * L, L)]
            out_v[pl.ds(sb * SB + blk * L, L)] = 1.0 / (1.0 + jnp.exp(-acc))
            return carry

        lax.fori_loop(0, NBLK, blk_body, 0)

    pltpu.sync_copy(out_v, out_hbm.at[pl.ds(base, BPW)])


@functools.partial(
    pl.kernel,
    out_type=jax.ShapeDtypeStruct((B,), jnp.float32),
    mesh=plsc.VectorSubcoreMesh(core_axis_name="c", subcore_axis_name="s"),
    compiler_params=pltpu.CompilerParams(needs_layout_passes=False,
                                         use_tc_tiling_on_sc=False),
    scratch_types=[
        pltpu.VMEM((BPW,), jnp.int32),
        pltpu.VMEM((BPW,), jnp.int32),
        pltpu.VMEM((BPW,), jnp.int32),
        pltpu.VMEM((BPW,), jnp.int32),
        pltpu.VMEM((BPW,), jnp.int32),
        pltpu.VMEM((BPW,), jnp.int32),
        pltpu.VMEM((SB, 128), jnp.float32),
        pltpu.VMEM((SB, 128), jnp.float32),
        pltpu.VMEM((D * H * L,), jnp.float32),
        pltpu.VMEM((H * L,), jnp.float32),
        pltpu.VMEM((H * L,), jnp.float32),
        pltpu.VMEM((L,), jnp.float32),
        pltpu.VMEM((BPW,), jnp.float32),
        pltpu.SemaphoreType.DMA,
    ],
)
def _gmf_kernel(*refs):
    _gmf_body(*refs)


def kernel(cell_indices, gene_indices, emb_cell, emb_gene, W1, b1, W2, b2):
    cell4 = emb_cell.reshape(-1, 128)
    gene4 = emb_gene.reshape(-1, 128)
    w1b = jnp.broadcast_to(W1.reshape(D, H, 1), (D, H, L)).reshape(-1)
    b1b = jnp.broadcast_to(b1.reshape(H, 1), (H, L)).reshape(-1)
    w2b = jnp.broadcast_to(W2.reshape(H, 1), (H, L)).reshape(-1)
    b2b = jnp.broadcast_to(b2.reshape(1, 1), (1, L)).reshape(-1)
    out = _gmf_kernel(cell_indices.astype(jnp.int32),
                      gene_indices.astype(jnp.int32),
                      cell4, gene4, w1b, b1b, w2b, b2b)
    return out.reshape(B, 1)


# speedup vs baseline: 1.0639x; 1.0179x over previous
import functools

import jax
import jax.numpy as jnp
from jax import lax
from jax.experimental import pallas as pl
from jax.experimental.pallas import tpu as pltpu
from jax.experimental.pallas import tpu_sc as plsc

D = 32
H = 16
B = 16384
L = 16
NC, NS = 2, 16
NW = NC * NS
BPW = B // NW
SB = 128
NSB = BPW // SB
NBLK2 = SB // (2 * L)


def _gmf_body(cell_idx_hbm, gene_idx_hbm, cell_tab, gene_tab,
              w1b_hbm, b1b_hbm, w2b_hbm, b2b_hbm, out_hbm,
              idx_c, idx_g, slot_c, slot_g, col_c, col_g,
              cell_b0, gene_b0, cell_b1, gene_b1,
              w1v, b1v, w2v, b2v, out_v, sem0, sem1):
    wid = lax.axis_index("s") * NC + lax.axis_index("c")
    base = wid * BPW

    pltpu.sync_copy(cell_idx_hbm.at[pl.ds(base, BPW)], idx_c)
    pltpu.sync_copy(gene_idx_hbm.at[pl.ds(base, BPW)], idx_g)
    pltpu.sync_copy(w1b_hbm, w1v)
    pltpu.sync_copy(b1b_hbm, b1v)
    pltpu.sync_copy(w2b_hbm, w2v)
    pltpu.sync_copy(b2b_hbm, b2v)

    for i in range(BPW // L):
        sl = pl.ds(i * L, L)
        ic = idx_c[sl]
        ig = idx_g[sl]
        slot_c[sl] = lax.shift_right_logical(ic, 2)
        slot_g[sl] = lax.shift_right_logical(ig, 2)
        col_c[sl] = lax.shift_left(jnp.bitwise_and(ic, 3), 5)
        col_g[sl] = lax.shift_left(jnp.bitwise_and(ig, 3), 5)

    lanes = lax.iota(jnp.int32, L)
    bufs = ((cell_b0, gene_b0, sem0), (cell_b1, gene_b1, sem1))

    def fire(sb):
        cb, gb, sem = bufs[sb % 2]
        src = pl.ds(sb * SB, SB)
        return (pltpu.async_copy(cell_tab.at[slot_c.at[src]], cb, sem),
                pltpu.async_copy(gene_tab.at[slot_g.at[src]], gb, sem))

    inflight = fire(0)
    for sb in range(NSB):
        cb, gb, _ = bufs[sb % 2]
        for cp in inflight:
            cp.wait()
        if sb + 1 < NSB:
            inflight = fire(sb + 1)

        def blk_body(j, carry, sb=sb, cb=cb, gb=gb):
            r0 = pl.ds(sb * SB + 2 * j * L, L)
            r1 = pl.ds(sb * SB + (2 * j + 1) * L, L)
            rows0 = lanes + 2 * j * L
            rows1 = rows0 + L
            cb0 = col_c[r0]
            cb1 = col_c[r1]
            gb0 = col_g[r0]
            gb1 = col_g[r1]
            h0 = [b1v[pl.ds(k * L, L)] for k in range(H)]
            h1 = list(h0)
            for d in range(D):
                p0 = (plsc.load_gather(cb, [rows0, cb0 + d])
                      * plsc.load_gather(gb, [rows0, gb0 + d]))
                p1 = (plsc.load_gather(cb, [rows1, cb1 + d])
                      * plsc.load_gather(gb, [rows1, gb1 + d]))
                for k in range(H):
                    w = w1v[pl.ds((d * H + k) * L, L)]
                    h0[k] = h0[k] + p0 * w
                    h1[k] = h1[k] + p1 * w
            acc0 = b2v[pl.ds(0, L)]
            acc1 = acc0
            for k in range(H):
                w = w2v[pl.ds(k * L, L)]
                acc0 = acc0 + jnp.maximum(h0[k], 0.0) * w
                acc1 = acc1 + jnp.maximum(h1[k], 0.0) * w
            out_v[r0] = 1.0 / (1.0 + jnp.exp(-acc0))
            out_v[r1] = 1.0 / (1.0 + jnp.exp(-acc1))
            return carry

        lax.fori_loop(0, NBLK2, blk_body, 0)

    pltpu.sync_copy(out_v, out_hbm.at[pl.ds(base, BPW)])


@functools.partial(
    pl.kernel,
    out_type=jax.ShapeDtypeStruct((B,), jnp.float32),
    mesh=plsc.VectorSubcoreMesh(core_axis_name="c", subcore_axis_name="s"),
    compiler_params=pltpu.CompilerParams(needs_layout_passes=False,
                                         use_tc_tiling_on_sc=False),
    scratch_types=[
        pltpu.VMEM((BPW,), jnp.int32),
        pltpu.VMEM((BPW,), jnp.int32),
        pltpu.VMEM((BPW,), jnp.int32),
        pltpu.VMEM((BPW,), jnp.int32),
        pltpu.VMEM((BPW,), jnp.int32),
        pltpu.VMEM((BPW,), jnp.int32),
        pltpu.VMEM((SB, 128), jnp.float32),
        pltpu.VMEM((SB, 128), jnp.float32),
        pltpu.VMEM((SB, 128), jnp.float32),
        pltpu.VMEM((SB, 128), jnp.float32),
        pltpu.VMEM((D * H * L,), jnp.float32),
        pltpu.VMEM((H * L,), jnp.float32),
        pltpu.VMEM((H * L,), jnp.float32),
        pltpu.VMEM((L,), jnp.float32),
        pltpu.VMEM((BPW,), jnp.float32),
        pltpu.SemaphoreType.DMA,
        pltpu.SemaphoreType.DMA,
    ],
)
def _gmf_kernel(*refs):
    _gmf_body(*refs)


def kernel(cell_indices, gene_indices, emb_cell, emb_gene, W1, b1, W2, b2):
    cell4 = emb_cell.reshape(-1, 128)
    gene4 = emb_gene.reshape(-1, 128)
    w1b = jnp.broadcast_to(W1.reshape(D, H, 1), (D, H, L)).reshape(-1)
    b1b = jnp.broadcast_to(b1.reshape(H, 1), (H, L)).reshape(-1)
    w2b = jnp.broadcast_to(W2.reshape(H, 1), (H, L)).reshape(-1)
    b2b = jnp.broadcast_to(b2.reshape(1, 1), (1, L)).reshape(-1)
    out = _gmf_kernel(cell_indices.astype(jnp.int32),
                      gene_indices.astype(jnp.int32),
                      cell4, gene4, w1b, b1b, w2b, b2b)
    return out.reshape(B, 1)
